# Initial kernel scaffold; baseline (speedup 1.0000x reference)
import jax
import jax.numpy as jnp
from jax.experimental import pallas as pl


def kernel(xs, candidates, persona, keys, values, label, shared_emb, cand_emb, R, R2):
    raise NotImplementedError("write your pallas kernel here")


# speedup vs baseline: 5.6541x; 5.6541x over previous
import functools

import jax
import jax.numpy as jnp
from jax import lax
from jax.experimental import pallas as pl
from jax.experimental.pallas import tpu as pltpu

VOCABP = 1024
EMB = 256
NKEY = 2048
NCAND = 1000
NPER = 128
ROWS = NKEY + NKEY + NCAND + NPER + 1
ROWSP = 5232
TOK = 64
SENTINEL = 1000
EPS = 1e-6

R_KEYS = 0
R_VALS = NKEY
R_CAND = 2 * NKEY
R_PERS = 2 * NKEY + NCAND
R_XS = 2 * NKEY + NCAND + NPER


def _counts_body(idx_ref, out_ref):
    rb = out_ref.shape[0]
    iota = lax.broadcasted_iota(jnp.int32, (rb, VOCABP), 1)
    acc = jnp.zeros((rb, VOCABP), jnp.float32)
    for t in range(TOK):
        col = idx_ref[:, t][:, None]
        acc = acc + (col == iota).astype(jnp.float32)
    out_ref[...] = acc


def _build_counts_tc(idx):
    rb = 16
    grid = ROWSP // rb
    return pl.pallas_call(
        _counts_body,
        grid=(grid,),
        in_specs=[pl.BlockSpec((rb, TOK), lambda i: (i, 0))],
        out_specs=pl.BlockSpec((rb, VOCABP), lambda i: (i, 0)),
        out_shape=jax.ShapeDtypeStruct((ROWSP, VOCABP), jnp.float32),
    )(idx)


def _softmax_row(x):
    m = jnp.max(x, axis=1, keepdims=True)
    e = jnp.exp(x - m)
    return e / jnp.sum(e, axis=1, keepdims=True)


def _row_norms(enc):
    n = jnp.sqrt(jnp.sum(enc * enc, axis=1))
    return jnp.maximum(n, EPS)[None, :]


def _vnorm(q):
    return jnp.maximum(jnp.sqrt(jnp.sum(q * q)), EPS)


def _dense_body(cnt_ref, e_ref, ce_ref, r_ref, r2_ref, out_ref):
    E = e_ref[...]
    f32 = jnp.float32
    enc_k = jnp.dot(cnt_ref[R_KEYS:R_KEYS + NKEY, :], E, preferred_element_type=f32)
    enc_v = jnp.dot(cnt_ref[R_VALS:R_VALS + NKEY, :], E, preferred_element_type=f32)
    enc_c = jnp.dot(cnt_ref[R_CAND:R_CAND + NCAND, :], ce_ref[...], preferred_element_type=f32)
    enc_p = jnp.dot(cnt_ref[R_PERS:R_PERS + NPER, :], E, preferred_element_type=f32)
    q = jnp.dot(cnt_ref[R_XS:R_XS + 1, :], E, preferred_element_type=f32)

    nk = _row_norms(enc_k)
    nc = _row_norms(enc_c)
    np_ = _row_norms(enc_p)

    def dotq(v, enc):
        return lax.dot_general(v, enc, (((1,), (1,)), ((), ())),
                               preferred_element_type=f32)

    for rm_ref in (r_ref, r2_ref):
        Rm = rm_ref[...]
        cos = dotq(q, enc_p) / (np_ * _vnorm(q))
        ret = _softmax_row(cos)
        hop = jnp.dot(ret, enc_p, preferred_element_type=f32)
        q_plus = dotq(q + hop, Rm)
        cos2 = dotq(q_plus, enc_k) / (nk * _vnorm(q_plus))
        ret2 = _softmax_row(cos2)
        hop2 = jnp.dot(ret2, enc_v, preferred_element_type=f32)
        q = dotq(q_plus + hop2, Rm)

    out_ref[...] = dotq(q, enc_c) / (nc * _vnorm(q))


def _dense(counts, epad, cepad, R, R2):
    return pl.pallas_call(
        _dense_body,
        out_shape=jax.ShapeDtypeStruct((1, NCAND), jnp.float32),
    )(counts, epad, cepad, R, R2)


def kernel(xs, candidates, persona, keys, values, label, shared_emb, cand_emb, R, R2):
    del label
    i32 = jnp.int32
    xs_pad = jnp.pad(xs.astype(i32), ((0, 0), (0, TOK - xs.shape[1])),
                     constant_values=SENTINEL)
    idx = jnp.concatenate([
        keys.astype(i32), values.astype(i32), candidates.astype(i32),
        persona.astype(i32), xs_pad,
    ], axis=0)
    idx = jnp.pad(idx, ((0, ROWSP - ROWS), (0, 0)), constant_values=SENTINEL)

    epad = jnp.pad(shared_emb, ((0, VOCABP - shared_emb.shape[0]), (0, 0)))
    cepad = jnp.pad(cand_emb, ((0, VOCABP - cand_emb.shape[0]), (0, 0)))

    counts = _build_counts_tc(idx)
    preds = _dense(counts, epad, cepad, R, R2)
    return preds.reshape(NCAND)


# speedup vs baseline: 13.0715x; 2.3119x over previous
import functools

import jax
import jax.numpy as jnp
from jax import lax
from jax.experimental import pallas as pl
from jax.experimental.pallas import tpu as pltpu
from jax.experimental.pallas import tpu_sc as plsc

VOCABP = 1024
EMB = 256
NKEY = 2048
NCAND = 1000
NPER = 128
ROWS = NKEY + NKEY + NCAND + NPER + 1
ROWSP = 5232
TOK = 64
SENTINEL = 1000
EPS = 1e-6

R_KEYS = 0
R_VALS = NKEY
R_CAND = 2 * NKEY
R_PERS = 2 * NKEY + NCAND
R_XS = 2 * NKEY + NCAND + NPER


GROUPS = ROWSP // 16


def _build_counts_sc(idx):
    nc, ns = 2, 16
    nw = nc * ns
    gpt = (GROUPS + nw - 1) // nw
    mesh = plsc.VectorSubcoreMesh(core_axis_name="c", subcore_axis_name="s")

    gidx = 16 * TOK
    gcnt = 16 * VOCABP

    @functools.partial(
        pl.kernel,
        mesh=mesh,
        compiler_params=pltpu.CompilerParams(needs_layout_passes=False),
        out_type=jax.ShapeDtypeStruct((ROWSP * VOCABP,), jnp.float32),
        scratch_types=[
            pltpu.VMEM((gidx,), jnp.int32),
            pltpu.VMEM((gcnt,), jnp.float32),
        ],
    )
    def k(idx_hbm, out_hbm, idx_v, cnt_v):
        wid = lax.axis_index("s") * nc + lax.axis_index("c")
        iota16 = lax.iota(jnp.int32, 16)
        row_tok = iota16 * TOK
        row_cnt = iota16 * VOCABP
        ones = jnp.ones((16,), jnp.float32)
        zeros = jnp.zeros((16,), jnp.float32)

        def zbody(j, _):
            cnt_v[pl.ds(j * 16, 16)] = zeros
            return 0
        lax.fori_loop(0, gcnt // 16, zbody, 0)

        def scatter_pass(val):
            def tbody(t, _):
                tok = plsc.load_gather(idx_v, [row_tok + t])
                plsc.addupdate_scatter(cnt_v, [row_cnt + tok], val)
                return 0
            lax.fori_loop(0, TOK, tbody, 0)

        for j in range(gpt):
            g = j * nw + wid

            @pl.when(g < GROUPS)
            def _():
                pltpu.sync_copy(idx_hbm.at[pl.ds(g * gidx, gidx)], idx_v)
                scatter_pass(ones)
                pltpu.sync_copy(cnt_v, out_hbm.at[pl.ds(g * gcnt, gcnt)])
                scatter_pass(-ones)

    return k(idx.reshape(ROWSP * TOK)).reshape(ROWSP, VOCABP)


def _softmax_row(x):
    m = jnp.max(x, axis=1, keepdims=True)
    e = jnp.exp(x - m)
    return e / jnp.sum(e, axis=1, keepdims=True)


def _row_norms(enc):
    n = jnp.sqrt(jnp.sum(enc * enc, axis=1))
    return jnp.maximum(n, EPS)[None, :]


def _vnorm(q):
    return jnp.maximum(jnp.sqrt(jnp.sum(q * q)), EPS)


def _dense_body(cnt_ref, e_ref, ce_ref, r_ref, r2_ref, out_ref):
    E = e_ref[...]
    f32 = jnp.float32
    enc_k = jnp.dot(cnt_ref[R_KEYS:R_KEYS + NKEY, :], E, preferred_element_type=f32)
    enc_v = jnp.dot(cnt_ref[R_VALS:R_VALS + NKEY, :], E, preferred_element_type=f32)
    enc_c = jnp.dot(cnt_ref[R_CAND:R_CAND + NCAND, :], ce_ref[...], preferred_element_type=f32)
    enc_p = jnp.dot(cnt_ref[R_PERS:R_PERS + NPER, :], E, preferred_element_type=f32)
    q = jnp.dot(cnt_ref[R_XS:R_XS + 1, :], E, preferred_element_type=f32)

    nk = _row_norms(enc_k)
    nc = _row_norms(enc_c)
    np_ = _row_norms(enc_p)

    def dotq(v, enc):
        return lax.dot_general(v, enc, (((1,), (1,)), ((), ())),
                               preferred_element_type=f32)

    for rm_ref in (r_ref, r2_ref):
        Rm = rm_ref[...]
        cos = dotq(q, enc_p) / (np_ * _vnorm(q))
        ret = _softmax_row(cos)
        hop = jnp.dot(ret, enc_p, preferred_element_type=f32)
        q_plus = dotq(q + hop, Rm)
        cos2 = dotq(q_plus, enc_k) / (nk * _vnorm(q_plus))
        ret2 = _softmax_row(cos2)
        hop2 = jnp.dot(ret2, enc_v, preferred_element_type=f32)
        q = dotq(q_plus + hop2, Rm)

    out_ref[...] = dotq(q, enc_c) / (nc * _vnorm(q))


def _dense(counts, epad, cepad, R, R2):
    return pl.pallas_call(
        _dense_body,
        out_shape=jax.ShapeDtypeStruct((1, NCAND), jnp.float32),
    )(counts, epad, cepad, R, R2)


def kernel(xs, candidates, persona, keys, values, label, shared_emb, cand_emb, R, R2):
    del label
    i32 = jnp.int32
    xs_pad = jnp.pad(xs.astype(i32), ((0, 0), (0, TOK - xs.shape[1])),
                     constant_values=SENTINEL)
    idx = jnp.concatenate([
        keys.astype(i32), values.astype(i32), candidates.astype(i32),
        persona.astype(i32), xs_pad,
    ], axis=0)
    idx = jnp.pad(idx, ((0, ROWSP - ROWS), (0, 0)), constant_values=SENTINEL)

    epad = jnp.pad(shared_emb, ((0, VOCABP - shared_emb.shape[0]), (0, 0)))
    cepad = jnp.pad(cand_emb, ((0, VOCABP - cand_emb.shape[0]), (0, 0)))

    counts = _build_counts_sc(idx)
    preds = _dense(counts, epad, cepad, R, R2)
    return preds.reshape(NCAND)


# speedup vs baseline: 16.1660x; 1.2367x over previous
import functools

import jax
import jax.numpy as jnp
from jax import lax
from jax.experimental import pallas as pl
from jax.experimental.pallas import tpu as pltpu
from jax.experimental.pallas import tpu_sc as plsc

VOCABP = 1024
EMB = 256
NKEY = 2048
NCAND = 1000
NPER = 128
ROWS = NKEY + NKEY + NCAND + NPER + 1
ROWSP = 5232
TOK = 64
SENTINEL = 1000
EPS = 1e-6

R_KEYS = 0
R_VALS = NKEY
R_CAND = 2 * NKEY
R_PERS = 2 * NKEY + NCAND
R_XS = 2 * NKEY + NCAND + NPER


GROUPS = ROWSP // 16


def _build_counts_sc(idx):
    nc, ns = 2, 16
    nw = nc * ns
    gpt = (GROUPS + nw - 1) // nw
    mesh = plsc.VectorSubcoreMesh(core_axis_name="c", subcore_axis_name="s")

    @functools.partial(
        pl.kernel,
        mesh=mesh,
        compiler_params=pltpu.CompilerParams(needs_layout_passes=False),
        out_type=jax.ShapeDtypeStruct((ROWSP, VOCABP), jnp.float32),
        scratch_types=[
            pltpu.VMEM((16, TOK), jnp.int32),
            pltpu.VMEM((16, VOCABP), jnp.float32),
        ],
    )
    def k(idx_hbm, out_hbm, idx_v, cnt_v):
        wid = lax.axis_index("s") * nc + lax.axis_index("c")
        iota16 = lax.iota(jnp.int32, 16)
        ones = jnp.ones((16,), jnp.float32)
        zeros = jnp.zeros((16,), jnp.float32)

        def zbody(j, _):
            cnt_v[j // (VOCABP // 16), pl.ds((j % (VOCABP // 16)) * 16, 16)] = zeros
            return 0
        lax.fori_loop(0, 16 * (VOCABP // 16), zbody, 0)

        def scatter_pass(val):
            def tbody(t, _):
                tvec = jnp.full((16,), t, jnp.int32)
                tok = plsc.load_gather(idx_v, [iota16, tvec])
                plsc.addupdate_scatter(cnt_v, [iota16, tok], val)
                return 0
            lax.fori_loop(0, TOK, tbody, 0)

        for j in range(gpt):
            g = j * nw + wid

            @pl.when(g < GROUPS)
            def _():
                pltpu.sync_copy(idx_hbm.at[pl.ds(g * 16, 16)], idx_v)
                scatter_pass(ones)
                pltpu.sync_copy(cnt_v, out_hbm.at[pl.ds(g * 16, 16)])
                scatter_pass(-ones)

    return k(idx)


def _softmax_row(x):
    m = jnp.max(x, axis=1, keepdims=True)
    e = jnp.exp(x - m)
    return e / jnp.sum(e, axis=1, keepdims=True)


def _row_norms(enc):
    n = jnp.sqrt(jnp.sum(enc * enc, axis=1))
    return jnp.maximum(n, EPS)[None, :]


def _vnorm(q):
    return jnp.maximum(jnp.sqrt(jnp.sum(q * q)), EPS)


def _dense_body(cnt_ref, e_ref, ce_ref, r_ref, r2_ref, out_ref):
    E = e_ref[...]
    f32 = jnp.float32
    enc_k = jnp.dot(cnt_ref[R_KEYS:R_KEYS + NKEY, :], E, preferred_element_type=f32)
    enc_v = jnp.dot(cnt_ref[R_VALS:R_VALS + NKEY, :], E, preferred_element_type=f32)
    enc_c = jnp.dot(cnt_ref[R_CAND:R_CAND + NCAND, :], ce_ref[...], preferred_element_type=f32)
    enc_p = jnp.dot(cnt_ref[R_PERS:R_PERS + NPER, :], E, preferred_element_type=f32)
    q = jnp.dot(cnt_ref[R_XS:R_XS + 1, :], E, preferred_element_type=f32)

    nk = _row_norms(enc_k)
    nc = _row_norms(enc_c)
    np_ = _row_norms(enc_p)

    def dotq(v, enc):
        return lax.dot_general(v, enc, (((1,), (1,)), ((), ())),
                               preferred_element_type=f32)

    for rm_ref in (r_ref, r2_ref):
        Rm = rm_ref[...]
        cos = dotq(q, enc_p) / (np_ * _vnorm(q))
        ret = _softmax_row(cos)
        hop = jnp.dot(ret, enc_p, preferred_element_type=f32)
        q_plus = dotq(q + hop, Rm)
        cos2 = dotq(q_plus, enc_k) / (nk * _vnorm(q_plus))
        ret2 = _softmax_row(cos2)
        hop2 = jnp.dot(ret2, enc_v, preferred_element_type=f32)
        q = dotq(q_plus + hop2, Rm)

    out_ref[...] = dotq(q, enc_c) / (nc * _vnorm(q))


def _dense(counts, epad, cepad, R, R2):
    return pl.pallas_call(
        _dense_body,
        out_shape=jax.ShapeDtypeStruct((1, NCAND), jnp.float32),
    )(counts, epad, cepad, R, R2)


def kernel(xs, candidates, persona, keys, values, label, shared_emb, cand_emb, R, R2):
    del label
    i32 = jnp.int32
    xs_pad = jnp.pad(xs.astype(i32), ((0, 0), (0, TOK - xs.shape[1])),
                     constant_values=SENTINEL)
    idx = jnp.concatenate([
        keys.astype(i32), values.astype(i32), candidates.astype(i32),
        persona.astype(i32), xs_pad,
    ], axis=0)
    idx = jnp.pad(idx, ((0, ROWSP - ROWS), (0, 0)), constant_values=SENTINEL)

    epad = jnp.pad(shared_emb, ((0, VOCABP - shared_emb.shape[0]), (0, 0)))
    cepad = jnp.pad(cand_emb, ((0, VOCABP - cand_emb.shape[0]), (0, 0)))

    counts = _build_counts_sc(idx)
    preds = _dense(counts, epad, cepad, R, R2)
    return preds.reshape(NCAND)


# speedup vs baseline: 18.5409x; 1.1469x over previous
import functools

import jax
import jax.numpy as jnp
from jax import lax
from jax.experimental import pallas as pl
from jax.experimental.pallas import tpu as pltpu
from jax.experimental.pallas import tpu_sc as plsc

VOCABP = 1024
EMB = 256
NKEY = 2048
NCAND = 1000
NPER = 128
ROWS = NKEY + NKEY + NCAND + NPER + 1
ROWSP = 5232
TOK = 64
SENTINEL = 1000
EPS = 1e-6

R_KEYS = 0
R_VALS = NKEY
R_CAND = 2 * NKEY
R_PERS = 2 * NKEY + NCAND
R_XS = 2 * NKEY + NCAND + NPER


GROUPS = ROWSP // 16


def _build_counts_sc(idx):
    nc, ns = 2, 16
    nw = nc * ns
    gpt = (GROUPS + nw - 1) // nw
    mesh = plsc.VectorSubcoreMesh(core_axis_name="c", subcore_axis_name="s")

    @functools.partial(
        pl.kernel,
        mesh=mesh,
        compiler_params=pltpu.CompilerParams(needs_layout_passes=False),
        out_type=jax.ShapeDtypeStruct((ROWSP, VOCABP), jnp.float32),
        scratch_types=[
            pltpu.VMEM((2, 16, TOK), jnp.int32),
            pltpu.VMEM((2, 16, VOCABP), jnp.float32),
            pltpu.SemaphoreType.DMA,
            pltpu.SemaphoreType.DMA,
        ],
    )
    def k(idx_hbm, out_hbm, idx_v, cnt_v, sem0, sem1):
        wid = lax.axis_index("s") * nc + lax.axis_index("c")
        iota16 = lax.iota(jnp.int32, 16)
        ones = jnp.ones((16,), jnp.float32)
        zeros = jnp.zeros((16,), jnp.float32)
        sems = (sem0, sem1)

        def zbody(j, _):
            for b in range(2):
                for r in range(16):
                    cnt_v[b, r, pl.ds(j * 16, 16)] = zeros
            return 0
        lax.fori_loop(0, VOCABP // 16, zbody, 0)

        def scatter_pass(b, val):
            def tbody(t, _):
                tvec = jnp.full((16,), t, jnp.int32)
                tok = plsc.load_gather(idx_v.at[b], [iota16, tvec])
                plsc.addupdate_scatter(cnt_v.at[b], [iota16, tok], val)
                return 0
            lax.fori_loop(0, TOK, tbody, 0, unroll=4)

        for j in range(gpt):
            g = j * nw + wid
            b = j % 2

            @pl.when(g < GROUPS)
            def _():
                if j >= 2:
                    gprev = (j - 2) * nw + wid
                    pltpu.make_async_copy(
                        cnt_v.at[b], out_hbm.at[pl.ds(gprev * 16, 16)], sems[b]
                    ).wait()
                    scatter_pass(b, -ones)
                pltpu.sync_copy(idx_hbm.at[pl.ds(g * 16, 16)], idx_v.at[b])
                scatter_pass(b, ones)
                pltpu.async_copy(
                    cnt_v.at[b], out_hbm.at[pl.ds(g * 16, 16)], sems[b]
                )

        for j in (gpt - 2, gpt - 1):
            g = j * nw + wid
            b = j % 2

            @pl.when(g < GROUPS)
            def _():
                pltpu.make_async_copy(
                    cnt_v.at[b], out_hbm.at[pl.ds(g * 16, 16)], sems[b]
                ).wait()

    return k(idx)


def _softmax_row(x):
    m = jnp.max(x, axis=1, keepdims=True)
    e = jnp.exp(x - m)
    return e / jnp.sum(e, axis=1, keepdims=True)


def _row_norms(enc):
    n = jnp.sqrt(jnp.sum(enc * enc, axis=1))
    return jnp.maximum(n, EPS)[None, :]


def _vnorm(q):
    return jnp.maximum(jnp.sqrt(jnp.sum(q * q)), EPS)


def _dense_body(cnt_ref, e_ref, ce_ref, r_ref, r2_ref, out_ref):
    E = e_ref[...]
    f32 = jnp.float32
    enc_k = jnp.dot(cnt_ref[R_KEYS:R_KEYS + NKEY, :], E, preferred_element_type=f32)
    enc_v = jnp.dot(cnt_ref[R_VALS:R_VALS + NKEY, :], E, preferred_element_type=f32)
    enc_c = jnp.dot(cnt_ref[R_CAND:R_CAND + NCAND, :], ce_ref[...], preferred_element_type=f32)
    enc_p = jnp.dot(cnt_ref[R_PERS:R_PERS + NPER, :], E, preferred_element_type=f32)
    q = jnp.dot(cnt_ref[R_XS:R_XS + 1, :], E, preferred_element_type=f32)

    nk = _row_norms(enc_k)
    nc = _row_norms(enc_c)
    np_ = _row_norms(enc_p)

    def dotq(v, enc):
        return lax.dot_general(v, enc, (((1,), (1,)), ((), ())),
                               preferred_element_type=f32)

    for rm_ref in (r_ref, r2_ref):
        Rm = rm_ref[...]
        cos = dotq(q, enc_p) / (np_ * _vnorm(q))
        ret = _softmax_row(cos)
        hop = jnp.dot(ret, enc_p, preferred_element_type=f32)
        q_plus = dotq(q + hop, Rm)
        cos2 = dotq(q_plus, enc_k) / (nk * _vnorm(q_plus))
        ret2 = _softmax_row(cos2)
        hop2 = jnp.dot(ret2, enc_v, preferred_element_type=f32)
        q = dotq(q_plus + hop2, Rm)

    out_ref[...] = dotq(q, enc_c) / (nc * _vnorm(q))


def _dense(counts, epad, cepad, R, R2):
    return pl.pallas_call(
        _dense_body,
        out_shape=jax.ShapeDtypeStruct((1, NCAND), jnp.float32),
    )(counts, epad, cepad, R, R2)


def kernel(xs, candidates, persona, keys, values, label, shared_emb, cand_emb, R, R2):
    del label
    i32 = jnp.int32
    xs_pad = jnp.pad(xs.astype(i32), ((0, 0), (0, TOK - xs.shape[1])),
                     constant_values=SENTINEL)
    idx = jnp.concatenate([
        keys.astype(i32), values.astype(i32), candidates.astype(i32),
        persona.astype(i32), xs_pad,
    ], axis=0)
    idx = jnp.pad(idx, ((0, ROWSP - ROWS), (0, 0)), constant_values=SENTINEL)

    epad = jnp.pad(shared_emb, ((0, VOCABP - shared_emb.shape[0]), (0, 0)))
    cepad = jnp.pad(cand_emb, ((0, VOCABP - cand_emb.shape[0]), (0, 0)))

    counts = _build_counts_sc(idx)
    preds = _dense(counts, epad, cepad, R, R2)
    return preds.reshape(NCAND)


# speedup vs baseline: 20.6198x; 1.1121x over previous
import functools

import jax
import jax.numpy as jnp
from jax import lax
from jax.experimental import pallas as pl
from jax.experimental.pallas import tpu as pltpu
from jax.experimental.pallas import tpu_sc as plsc

VOCABP = 1024
EMB = 256
NKEY = 2048
NCAND = 1000
NPER = 128
ROWS = NKEY + NKEY + NCAND + NPER + 1
ROWSP = 5232
TOK = 64
SENTINEL = 1000
EPS = 1e-6

R_KEYS = 0
R_VALS = NKEY
R_CAND = 2 * NKEY
R_PERS = 2 * NKEY + NCAND
R_XS = 2 * NKEY + NCAND + NPER


GROUPS = ROWSP // 16


def _build_counts_sc(idx):
    nc, ns = 2, 16
    nw = nc * ns
    gpt = (GROUPS + nw - 1) // nw
    mesh = plsc.VectorSubcoreMesh(core_axis_name="c", subcore_axis_name="s")

    @functools.partial(
        pl.kernel,
        mesh=mesh,
        compiler_params=pltpu.CompilerParams(needs_layout_passes=False),
        out_type=jax.ShapeDtypeStruct((ROWSP, VOCABP), jnp.float32),
        scratch_types=[
            pltpu.VMEM((4, 16, TOK), jnp.int32),
            pltpu.VMEM((2, 16, VOCABP), jnp.float32),
            pltpu.SemaphoreType.DMA,
            pltpu.SemaphoreType.DMA,
            pltpu.SemaphoreType.DMA,
            pltpu.SemaphoreType.DMA,
            pltpu.SemaphoreType.DMA,
            pltpu.SemaphoreType.DMA,
        ],
    )
    def k(idx_hbm, out_hbm, idx_v, cnt_v, os0, os1, is0, is1, is2, is3):
        wid = lax.axis_index("s") * nc + lax.axis_index("c")
        iota16 = lax.iota(jnp.int32, 16)
        ones = jnp.ones((16,), jnp.float32)
        zeros = jnp.zeros((16,), jnp.float32)
        osems = (os0, os1)
        isems = (is0, is1, is2, is3)

        def idx_fetch(j, start):
            g = j * nw + wid
            s = j % 4

            @pl.when(g < GROUPS)
            def _():
                cp = (pltpu.async_copy if start else pltpu.make_async_copy)(
                    idx_hbm.at[pl.ds(g * 16, 16)], idx_v.at[s], isems[s])
                if not start:
                    cp.wait()

        idx_fetch(0, True)

        def zbody(j, _):
            for b in range(2):
                for r in range(16):
                    cnt_v[b, r, pl.ds(j * 16, 16)] = zeros
            return 0
        lax.fori_loop(0, VOCABP // 16, zbody, 0)

        def scatter_pass(s, b, val):
            def tbody(t, _):
                tvec = jnp.full((16,), t, jnp.int32)
                tok = plsc.load_gather(idx_v.at[s], [iota16, tvec])
                plsc.addupdate_scatter(cnt_v.at[b], [iota16, tok], val)
                return 0
            lax.fori_loop(0, TOK, tbody, 0, unroll=8)

        for j in range(gpt):
            g = j * nw + wid
            b = j % 2
            if j + 1 < gpt:
                idx_fetch(j + 1, True)

            @pl.when(g < GROUPS)
            def _():
                if j >= 2:
                    gprev = (j - 2) * nw + wid
                    pltpu.make_async_copy(
                        cnt_v.at[b], out_hbm.at[pl.ds(gprev * 16, 16)], osems[b]
                    ).wait()
                    scatter_pass((j - 2) % 4, b, -ones)
            idx_fetch(j, False)

            @pl.when(g < GROUPS)
            def _():
                scatter_pass(j % 4, b, ones)
                pltpu.async_copy(
                    cnt_v.at[b], out_hbm.at[pl.ds(g * 16, 16)], osems[b]
                )

        for j in (gpt - 2, gpt - 1):
            g = j * nw + wid
            b = j % 2

            @pl.when(g < GROUPS)
            def _():
                pltpu.make_async_copy(
                    cnt_v.at[b], out_hbm.at[pl.ds(g * 16, 16)], osems[b]
                ).wait()

    return k(idx)


def _softmax_row(x):
    m = jnp.max(x, axis=1, keepdims=True)
    e = jnp.exp(x - m)
    return e / jnp.sum(e, axis=1, keepdims=True)


def _row_norms(enc):
    n = jnp.sqrt(jnp.sum(enc * enc, axis=1))
    return jnp.maximum(n, EPS)[None, :]


def _vnorm(q):
    return jnp.maximum(jnp.sqrt(jnp.sum(q * q)), EPS)


def _dense_body(cnt_ref, e_ref, ce_ref, r_ref, r2_ref, out_ref):
    E = e_ref[...]
    f32 = jnp.float32
    enc_k = jnp.dot(cnt_ref[R_KEYS:R_KEYS + NKEY, :], E, preferred_element_type=f32)
    enc_v = jnp.dot(cnt_ref[R_VALS:R_VALS + NKEY, :], E, preferred_element_type=f32)
    enc_c = jnp.dot(cnt_ref[R_CAND:R_CAND + NCAND, :], ce_ref[...], preferred_element_type=f32)
    enc_p = jnp.dot(cnt_ref[R_PERS:R_PERS + NPER, :], E, preferred_element_type=f32)
    q = jnp.dot(cnt_ref[R_XS:R_XS + 1, :], E, preferred_element_type=f32)

    nk = _row_norms(enc_k)
    nc = _row_norms(enc_c)
    np_ = _row_norms(enc_p)

    def dotq(v, enc):
        return lax.dot_general(v, enc, (((1,), (1,)), ((), ())),
                               preferred_element_type=f32)

    for rm_ref in (r_ref, r2_ref):
        Rm = rm_ref[...]
        cos = dotq(q, enc_p) / (np_ * _vnorm(q))
        ret = _softmax_row(cos)
        hop = jnp.dot(ret, enc_p, preferred_element_type=f32)
        q_plus = dotq(q + hop, Rm)
        cos2 = dotq(q_plus, enc_k) / (nk * _vnorm(q_plus))
        ret2 = _softmax_row(cos2)
        hop2 = jnp.dot(ret2, enc_v, preferred_element_type=f32)
        q = dotq(q_plus + hop2, Rm)

    out_ref[...] = dotq(q, enc_c) / (nc * _vnorm(q))


def _dense(counts, epad, cepad, R, R2):
    return pl.pallas_call(
        _dense_body,
        out_shape=jax.ShapeDtypeStruct((1, NCAND), jnp.float32),
    )(counts, epad, cepad, R, R2)


def kernel(xs, candidates, persona, keys, values, label, shared_emb, cand_emb, R, R2):
    del label
    i32 = jnp.int32
    xs_pad = jnp.pad(xs.astype(i32), ((0, 0), (0, TOK - xs.shape[1])),
                     constant_values=SENTINEL)
    idx = jnp.concatenate([
        keys.astype(i32), values.astype(i32), candidates.astype(i32),
        persona.astype(i32), xs_pad,
    ], axis=0)
    idx = jnp.pad(idx, ((0, ROWSP - ROWS), (0, 0)), constant_values=SENTINEL)

    epad = jnp.pad(shared_emb, ((0, VOCABP - shared_emb.shape[0]), (0, 0)))
    cepad = jnp.pad(cand_emb, ((0, VOCABP - cand_emb.shape[0]), (0, 0)))

    counts = _build_counts_sc(idx)
    preds = _dense(counts, epad, cepad, R, R2)
    return preds.reshape(NCAND)


# speedup vs baseline: 21.1852x; 1.0274x over previous
import functools

import jax
import jax.numpy as jnp
from jax import lax
from jax.experimental import pallas as pl
from jax.experimental.pallas import tpu as pltpu
from jax.experimental.pallas import tpu_sc as plsc

VOCABP = 1024
EMB = 256
NKEY = 2048
NCAND = 1000
NPER = 128
ROWS = NKEY + NKEY + NCAND + NPER + 1
ROWSP = 5232
TOK = 64
SENTINEL = 1000
EPS = 1e-6

R_KEYS = 0
R_VALS = NKEY
R_CAND = 2 * NKEY
R_PERS = 2 * NKEY + NCAND
R_XS = 2 * NKEY + NCAND + NPER


GROUPS = ROWSP // 16


def _build_counts_sc(idx):
    nc, ns = 2, 16
    nw = nc * ns
    gpt = (GROUPS + nw - 1) // nw
    mesh = plsc.VectorSubcoreMesh(core_axis_name="c", subcore_axis_name="s")

    @functools.partial(
        pl.kernel,
        mesh=mesh,
        compiler_params=pltpu.CompilerParams(needs_layout_passes=False),
        out_type=jax.ShapeDtypeStruct((ROWSP, VOCABP), jnp.float32),
        scratch_types=[
            pltpu.VMEM((4, 16, TOK), jnp.int32),
            pltpu.VMEM((2, 16, VOCABP), jnp.float32),
            pltpu.SemaphoreType.DMA,
            pltpu.SemaphoreType.DMA,
            pltpu.SemaphoreType.DMA,
            pltpu.SemaphoreType.DMA,
            pltpu.SemaphoreType.DMA,
            pltpu.SemaphoreType.DMA,
        ],
    )
    def k(idx_hbm, out_hbm, idx_v, cnt_v, os0, os1, is0, is1, is2, is3):
        wid = lax.axis_index("s") * nc + lax.axis_index("c")
        iota16 = lax.iota(jnp.int32, 16)
        ones = jnp.ones((16,), jnp.float32)
        zeros = jnp.zeros((16,), jnp.float32)
        osems = (os0, os1)
        isems = (is0, is1, is2, is3)

        def idx_fetch(g, s, start):
            @pl.when(g < GROUPS)
            def _():
                cp = (pltpu.async_copy if start else pltpu.make_async_copy)(
                    idx_hbm.at[pl.ds(g * 16, 16)], idx_v.at[s], isems[s])
                if not start:
                    cp.wait()

        idx_fetch(wid, 0, True)

        def zbody(j, _):
            for b in range(2):
                for r in range(16):
                    cnt_v[b, r, pl.ds(j * 16, 16)] = zeros
            return 0
        lax.fori_loop(0, VOCABP // 16, zbody, 0)

        def scatter_pass(s, b, val):
            def tbody(t, _):
                tvec = jnp.full((16,), t, jnp.int32)
                tok = plsc.load_gather(idx_v.at[s], [iota16, tvec])
                plsc.addupdate_scatter(cnt_v.at[b], [iota16, tok], val)
                return 0
            lax.fori_loop(0, TOK, tbody, 0, unroll=8)

        nblk = -(-gpt // 4)

        def block(i, _):
            for u in range(4):
                jj = i * 4 + u
                g = jj * nw + wid
                b = u % 2
                idx_fetch(g + nw, (u + 1) % 4, True)

                @pl.when((g < GROUPS) & (jj >= 2))
                def _():
                    gprev = g - 2 * nw
                    pltpu.make_async_copy(
                        cnt_v.at[b], out_hbm.at[pl.ds(gprev * 16, 16)], osems[b]
                    ).wait()
                    scatter_pass((u + 2) % 4, b, -ones)
                idx_fetch(g, u, False)

                @pl.when(g < GROUPS)
                def _():
                    scatter_pass(u, b, ones)
                    pltpu.async_copy(
                        cnt_v.at[b], out_hbm.at[pl.ds(g * 16, 16)], osems[b]
                    )
            return 0

        lax.fori_loop(0, nblk, block, 0)

        for b in range(2):
            pltpu.make_async_copy(
                cnt_v.at[b], out_hbm.at[pl.ds(0, 16)], osems[b]
            ).wait()

    return k(idx)


def _softmax_row(x):
    m = jnp.max(x, axis=1, keepdims=True)
    e = jnp.exp(x - m)
    return e / jnp.sum(e, axis=1, keepdims=True)


def _row_norms(enc):
    n = jnp.sqrt(jnp.sum(enc * enc, axis=1))
    return jnp.maximum(n, EPS)[None, :]


def _vnorm(q):
    return jnp.maximum(jnp.sqrt(jnp.sum(q * q)), EPS)


def _dense_body(cnt_ref, e_ref, ce_ref, r_ref, r2_ref, out_ref):
    E = e_ref[...]
    f32 = jnp.float32
    enc_k = jnp.dot(cnt_ref[R_KEYS:R_KEYS + NKEY, :], E, preferred_element_type=f32)
    enc_v = jnp.dot(cnt_ref[R_VALS:R_VALS + NKEY, :], E, preferred_element_type=f32)
    enc_c = jnp.dot(cnt_ref[R_CAND:R_CAND + NCAND, :], ce_ref[...], preferred_element_type=f32)
    enc_p = jnp.dot(cnt_ref[R_PERS:R_PERS + NPER, :], E, preferred_element_type=f32)
    q = jnp.dot(cnt_ref[R_XS:R_XS + 1, :], E, preferred_element_type=f32)

    nk = _row_norms(enc_k)
    nc = _row_norms(enc_c)
    np_ = _row_norms(enc_p)

    def dotq(v, enc):
        return lax.dot_general(v, enc, (((1,), (1,)), ((), ())),
                               preferred_element_type=f32)

    for rm_ref in (r_ref, r2_ref):
        Rm = rm_ref[...]
        cos = dotq(q, enc_p) / (np_ * _vnorm(q))
        ret = _softmax_row(cos)
        hop = jnp.dot(ret, enc_p, preferred_element_type=f32)
        q_plus = dotq(q + hop, Rm)
        cos2 = dotq(q_plus, enc_k) / (nk * _vnorm(q_plus))
        ret2 = _softmax_row(cos2)
        hop2 = jnp.dot(ret2, enc_v, preferred_element_type=f32)
        q = dotq(q_plus + hop2, Rm)

    out_ref[...] = dotq(q, enc_c) / (nc * _vnorm(q))


def _dense(counts, epad, cepad, R, R2):
    return pl.pallas_call(
        _dense_body,
        out_shape=jax.ShapeDtypeStruct((1, NCAND), jnp.float32),
    )(counts, epad, cepad, R, R2)


def kernel(xs, candidates, persona, keys, values, label, shared_emb, cand_emb, R, R2):
    del label
    i32 = jnp.int32
    xs_pad = jnp.pad(xs.astype(i32), ((0, 0), (0, TOK - xs.shape[1])),
                     constant_values=SENTINEL)
    idx = jnp.concatenate([
        keys.astype(i32), values.astype(i32), candidates.astype(i32),
        persona.astype(i32), xs_pad,
    ], axis=0)
    idx = jnp.pad(idx, ((0, ROWSP - ROWS), (0, 0)), constant_values=SENTINEL)

    epad = jnp.pad(shared_emb, ((0, VOCABP - shared_emb.shape[0]), (0, 0)))
    cepad = jnp.pad(cand_emb, ((0, VOCABP - cand_emb.shape[0]), (0, 0)))

    counts = _build_counts_sc(idx)
    preds = _dense(counts, epad, cepad, R, R2)
    return preds.reshape(NCAND)
